# baseline (speedup 1.0000x reference)
import functools

import jax
import jax.numpy as jnp
from jax.experimental import pallas as pl
from jax.experimental.pallas import tpu as pltpu

B = 64
DIM_CODES = 32
BOOK_SIZE = 8192
EMB = 32
W_DIM = DIM_CODES * EMB
Z_DIM = 256
N_PSEUDO = 500
H_DIM = 512

N_ROWS = B + N_PSEUDO
N_PAD = 568
KT = 512
NK = BOOK_SIZE // KT
NC = 1
DCC = DIM_CODES // NC
NBUF = 3
NPRE = 3
NCH = KT // 128


def _fused_kernel(xcat_ref, cb_ref, We1_ref, be1_ref, We2_ref, be2_ref,
                  Wd1_ref, bd1_ref, Wd2_ref, bd2_ref, eps_ref,
                  enc_ref, z_ref, cwr_ref, dist_ref, idx_ref,
                  xqs_s, x2c_s, rminv_s, rmini_s, dbuf_s, cbuf_s, sem, sem_in):
    c = pl.program_id(0)
    k = pl.program_id(1)
    slot = jax.lax.rem(k, NBUF)

    def _fetch(t, sl):
        pltpu.make_async_copy(
            cb_ref.at[:, pl.ds(t * KT, KT), :],
            cbuf_s.at[sl],
            sem_in.at[sl]).start()

    @pl.when(k == 0)
    def _prefetch0():
        for t in range(NPRE):
            _fetch(t, t)

    @pl.when(k == 0)
    def _mlp():
        h = jnp.maximum(xcat_ref[...] @ We1_ref[...] + be1_ref[...], 0.0)
        enc = h @ We2_ref[...] + be2_ref[...]
        enc_ref[0] = enc
        mu = enc[:B, :Z_DIM]
        log_var = enc[:B, Z_DIM:]
        z = eps_ref[...] * jnp.exp(0.5 * log_var) + mu
        z_ref[0] = z
        d = jnp.maximum(z @ Wd1_ref[...] + bd1_ref[...], 0.0)
        cwr = d @ Wd2_ref[...] + bd2_ref[...]
        cwr_ref[0] = cwr
        xqs = jnp.transpose(cwr.reshape(B, DIM_CODES, EMB), (1, 0, 2))
        x2c_s[...] = jnp.sum(xqs * xqs, axis=2, keepdims=True)
        xqs_s[...] = -2.0 * xqs
        rminv_s[...] = jnp.full((DIM_CODES, B, 128), jnp.inf, jnp.float32)
        rmini_s[...] = jnp.zeros((DIM_CODES, B, 128), jnp.int32)

    @pl.when(k >= NBUF)
    def _wait_prev():
        old = (k - NBUF) * KT
        for dd in range(DCC):
            pltpu.make_async_copy(
                dbuf_s.at[slot, dd],
                dist_ref.at[:, c * DCC + dd, pl.ds(old, KT)],
                sem.at[slot, dd]).wait()

    islot = jax.lax.rem(k, NPRE)
    pltpu.make_async_copy(
        cb_ref.at[:, pl.ds(k * KT, KT), :],
        cbuf_s.at[islot],
        sem_in.at[islot]).wait()
    cbT = jnp.transpose(cbuf_s[islot], (0, 2, 1))
    y2 = jnp.sum(cbT * cbT, axis=1, keepdims=True)
    base = k * KT
    for dd in range(DCC):
        da = c * DCC + dd
        xy = jnp.dot(xqs_s[da], cbT[dd])
        dist_d = (x2c_s[da] + y2[dd]) + xy
        dbuf_s[slot, dd] = dist_d
        m = dist_d[:, 0:128]
        mi = jnp.full((B, 128), base, jnp.int32)
        for ch in range(1, NCH):
            cm = dist_d[:, ch * 128:(ch + 1) * 128]
            upd = cm < m
            m = jnp.where(upd, cm, m)
            mi = jnp.where(upd, base + ch * 128, mi)
        rv = rminv_s[da]
        upd2 = m < rv
        rminv_s[da] = jnp.where(upd2, m, rv)
        rmini_s[da] = jnp.where(upd2, mi, rmini_s[da])
        pltpu.make_async_copy(
            dbuf_s.at[slot, dd],
            dist_ref.at[:, da, pl.ds(base, KT)],
            sem.at[slot, dd]).start()

    @pl.when(k + NPRE <= NK - 1)
    def _refetch():
        _fetch(k + NPRE, islot)

    @pl.when(k == NK - 1)
    def _finish():
        for kk in range(NK - NBUF, NK):
            off = kk * KT
            ss = kk % NBUF
            for dd in range(DCC):
                pltpu.make_async_copy(
                    dbuf_s.at[ss, dd],
                    dist_ref.at[:, c * DCC + dd, pl.ds(off, KT)],
                    sem.at[ss, dd]).wait()
        lanes = jax.lax.broadcasted_iota(jnp.int32, (B, 128), 1)
        cols = []
        for dd in range(DCC):
            v = rminv_s[c * DCC + dd]
            gmin = jnp.min(v, axis=1, keepdims=True)
            gidx = rmini_s[c * DCC + dd] + lanes
            cand = jnp.where(v == gmin, gidx, jnp.int32(2**31 - 1))
            cols.append(jnp.min(cand, axis=1)[:, None])
        idx_ref[0] = jnp.concatenate(cols, axis=1)


@functools.partial(jax.jit, static_argnums=())
def _run(xcat, codebook, We1, be1, We2, be2, Wd1, bd1, Wd2, bd2, eps):
    full = lambda shape: pl.BlockSpec(shape, lambda c, k: (0,) * len(shape))
    out_shapes = (
        jax.ShapeDtypeStruct((NC, N_PAD, 2 * Z_DIM), jnp.float32),
        jax.ShapeDtypeStruct((NC, B, Z_DIM), jnp.float32),
        jax.ShapeDtypeStruct((NC, B, W_DIM), jnp.float32),
        jax.ShapeDtypeStruct((B, DIM_CODES, BOOK_SIZE), jnp.float32),
        jax.ShapeDtypeStruct((NC, B, DCC), jnp.int32),
    )
    percore = lambda shape: pl.BlockSpec(
        (1,) + shape, lambda c, k: (c,) + (0,) * len(shape))
    return pl.pallas_call(
        _fused_kernel,
        grid=(NC, NK),
        in_specs=[
            full((N_PAD, W_DIM)),
            pl.BlockSpec(memory_space=pl.ANY),
            full((W_DIM, H_DIM)),
            full((1, H_DIM)),
            full((H_DIM, 2 * Z_DIM)),
            full((1, 2 * Z_DIM)),
            full((Z_DIM, H_DIM)),
            full((1, H_DIM)),
            full((H_DIM, W_DIM)),
            full((1, W_DIM)),
            full((B, Z_DIM)),
        ],
        out_specs=(
            percore((N_PAD, 2 * Z_DIM)),
            percore((B, Z_DIM)),
            percore((B, W_DIM)),
            pl.BlockSpec(memory_space=pl.ANY),
            percore((B, DCC)),
        ),
        out_shape=out_shapes,
        scratch_shapes=[
            pltpu.VMEM((DIM_CODES, B, EMB), jnp.float32),
            pltpu.VMEM((DIM_CODES, B, 1), jnp.float32),
            pltpu.VMEM((DIM_CODES, B, 128), jnp.float32),
            pltpu.VMEM((DIM_CODES, B, 128), jnp.int32),
            pltpu.VMEM((NBUF, DCC, B, KT), jnp.float32),
            pltpu.VMEM((NPRE, DIM_CODES, KT, EMB), jnp.float32),
            pltpu.SemaphoreType.DMA((NBUF, DCC)),
            pltpu.SemaphoreType.DMA((NPRE,)),
        ],
        compiler_params=pltpu.CompilerParams(
            dimension_semantics=("parallel", "arbitrary"),
        ),
    )(xcat, codebook, We1, be1, We2, be2, Wd1, bd1, Wd2, bd2, eps)


def kernel(x, codebook, pseudo_inputs, We1, be1, We2, be2, Wd1, bd1, Wd2, bd2, eps):
    xr = x.reshape(B, DIM_CODES, EMB).transpose(0, 2, 1).reshape(B, W_DIM)
    pr = pseudo_inputs.reshape(N_PSEUDO, W_DIM)
    xcat = jnp.concatenate(
        [xr, pr, jnp.zeros((N_PAD - N_ROWS, W_DIM), jnp.float32)], axis=0)
    enc2, z2, cwr2, cw_dist, idx2 = _run(
        xcat, codebook,
        We1, be1.reshape(1, -1), We2, be2.reshape(1, -1),
        Wd1, bd1.reshape(1, -1), Wd2, bd2.reshape(1, -1), eps)
    enc = enc2[0]
    mu = enc[:B, :Z_DIM]
    log_var = enc[:B, Z_DIM:]
    pseudo_mu = enc[B:N_ROWS, :Z_DIM]
    pseudo_log_var = enc[B:N_ROWS, Z_DIM:]
    idx = jnp.concatenate([idx2[i] for i in range(NC)], axis=1)
    return (mu, log_var, pseudo_mu, pseudo_log_var, z2[0], cwr2[0], cw_dist,
            idx.reshape(B, DIM_CODES, 1))


# speedup vs baseline: 1.1611x; 1.1611x over previous
import functools

import jax
import jax.numpy as jnp
from jax.experimental import pallas as pl
from jax.experimental.pallas import tpu as pltpu

B = 64
DIM_CODES = 32
BOOK_SIZE = 8192
EMB = 32
W_DIM = DIM_CODES * EMB
Z_DIM = 256
N_PSEUDO = 500
H_DIM = 512

N_ROWS = B + N_PSEUDO
N_PAD = 568
KT = 512
NK = BOOK_SIZE // KT
NC = 1
DCC = DIM_CODES // NC
NBUF = 3
NPRE = 3
NCH = KT // 128


def _fused_kernel(xcat_ref, cb_ref, We1_ref, be1_ref, We2_ref, be2_ref,
                  Wd1_ref, bd1_ref, Wd2_ref, bd2_ref, eps_ref,
                  enc_ref, z_ref, cwr_ref, dist_ref, idx_ref,
                  xqs_s, x2c_s, rminv_s, rmini_s, dbuf_s, cbuf_s, sem, sem_in):
    c = pl.program_id(0)
    k = pl.program_id(1)
    slot = jax.lax.rem(k, NBUF)

    def _fetch(t, sl):
        pltpu.make_async_copy(
            cb_ref.at[:, pl.ds(t * KT, KT), :],
            cbuf_s.at[sl],
            sem_in.at[sl]).start()

    @pl.when(k == 0)
    def _prefetch0():
        for t in range(NPRE):
            _fetch(t, t)

    @pl.when(k == 0)
    def _mlp():
        h = jnp.maximum(xcat_ref[...] @ We1_ref[...] + be1_ref[...], 0.0)
        enc = h @ We2_ref[...] + be2_ref[...]
        enc_ref[0] = enc
        mu = enc[:B, :Z_DIM]
        log_var = enc[:B, Z_DIM:]
        z = eps_ref[...] * jnp.exp(0.5 * log_var) + mu
        z_ref[0] = z
        d = jnp.maximum(z @ Wd1_ref[...] + bd1_ref[...], 0.0)
        cwr = d @ Wd2_ref[...] + bd2_ref[...]
        cwr_ref[0] = cwr
        xqs = jnp.transpose(cwr.reshape(B, DIM_CODES, EMB), (1, 0, 2))
        x2c_s[...] = jnp.sum(xqs * xqs, axis=2, keepdims=True)
        xqs_s[...] = -2.0 * xqs
        rminv_s[...] = jnp.full((DIM_CODES, B, 128), jnp.inf, jnp.float32)
        rmini_s[...] = jnp.zeros((DIM_CODES, B, 128), jnp.int32)

    @pl.when(k >= NBUF)
    def _wait_prev():
        old = (k - NBUF) * KT
        for dd in range(DCC):
            pltpu.make_async_copy(
                dbuf_s.at[slot, dd],
                dist_ref.at[:, c * DCC + dd, pl.ds(old, KT)],
                sem.at[slot, dd]).wait()

    islot = jax.lax.rem(k, NPRE)
    pltpu.make_async_copy(
        cb_ref.at[:, pl.ds(k * KT, KT), :],
        cbuf_s.at[islot],
        sem_in.at[islot]).wait()
    cbT = jnp.transpose(cbuf_s[islot], (0, 2, 1))
    y2 = jnp.sum(cbT * cbT, axis=1, keepdims=True)
    base = k * KT
    for dd in range(DCC):
        da = c * DCC + dd
        xy = jnp.dot(xqs_s[da], cbT[dd])
        dist_d = (x2c_s[da] + y2[dd]) + xy
        dbuf_s[slot, dd] = dist_d
        m = dist_d[:, 0:128]
        mi = jnp.full((B, 128), base, jnp.int32)
        for ch in range(1, NCH):
            cm = dist_d[:, ch * 128:(ch + 1) * 128]
            upd = cm < m
            m = jnp.where(upd, cm, m)
            mi = jnp.where(upd, base + ch * 128, mi)
        rv = rminv_s[da]
        upd2 = m < rv
        rminv_s[da] = jnp.where(upd2, m, rv)
        rmini_s[da] = jnp.where(upd2, mi, rmini_s[da])

    for dd in range(DCC):
        pltpu.make_async_copy(
            dbuf_s.at[slot, dd],
            dist_ref.at[:, c * DCC + dd, pl.ds(base, KT)],
            sem.at[slot, dd]).start()

    @pl.when(k + NPRE <= NK - 1)
    def _refetch():
        _fetch(k + NPRE, islot)

    @pl.when(k == NK - 1)
    def _finish():
        for kk in range(NK - NBUF, NK):
            off = kk * KT
            ss = kk % NBUF
            for dd in range(DCC):
                pltpu.make_async_copy(
                    dbuf_s.at[ss, dd],
                    dist_ref.at[:, c * DCC + dd, pl.ds(off, KT)],
                    sem.at[ss, dd]).wait()
        lanes = jax.lax.broadcasted_iota(jnp.int32, (B, 128), 1)
        cols = []
        for dd in range(DCC):
            v = rminv_s[c * DCC + dd]
            gmin = jnp.min(v, axis=1, keepdims=True)
            gidx = rmini_s[c * DCC + dd] + lanes
            cand = jnp.where(v == gmin, gidx, jnp.int32(2**31 - 1))
            cols.append(jnp.min(cand, axis=1)[:, None])
        idx_ref[0] = jnp.concatenate(cols, axis=1)


@functools.partial(jax.jit, static_argnums=())
def _run(xcat, codebook, We1, be1, We2, be2, Wd1, bd1, Wd2, bd2, eps):
    full = lambda shape: pl.BlockSpec(shape, lambda c, k: (0,) * len(shape))
    out_shapes = (
        jax.ShapeDtypeStruct((NC, N_PAD, 2 * Z_DIM), jnp.float32),
        jax.ShapeDtypeStruct((NC, B, Z_DIM), jnp.float32),
        jax.ShapeDtypeStruct((NC, B, W_DIM), jnp.float32),
        jax.ShapeDtypeStruct((B, DIM_CODES, BOOK_SIZE), jnp.float32),
        jax.ShapeDtypeStruct((NC, B, DCC), jnp.int32),
    )
    percore = lambda shape: pl.BlockSpec(
        (1,) + shape, lambda c, k: (c,) + (0,) * len(shape))
    return pl.pallas_call(
        _fused_kernel,
        grid=(NC, NK),
        in_specs=[
            full((N_PAD, W_DIM)),
            pl.BlockSpec(memory_space=pl.ANY),
            full((W_DIM, H_DIM)),
            full((1, H_DIM)),
            full((H_DIM, 2 * Z_DIM)),
            full((1, 2 * Z_DIM)),
            full((Z_DIM, H_DIM)),
            full((1, H_DIM)),
            full((H_DIM, W_DIM)),
            full((1, W_DIM)),
            full((B, Z_DIM)),
        ],
        out_specs=(
            percore((N_PAD, 2 * Z_DIM)),
            percore((B, Z_DIM)),
            percore((B, W_DIM)),
            pl.BlockSpec(memory_space=pl.ANY),
            percore((B, DCC)),
        ),
        out_shape=out_shapes,
        scratch_shapes=[
            pltpu.VMEM((DIM_CODES, B, EMB), jnp.float32),
            pltpu.VMEM((DIM_CODES, B, 1), jnp.float32),
            pltpu.VMEM((DIM_CODES, B, 128), jnp.float32),
            pltpu.VMEM((DIM_CODES, B, 128), jnp.int32),
            pltpu.VMEM((NBUF, DCC, B, KT), jnp.float32),
            pltpu.VMEM((NPRE, DIM_CODES, KT, EMB), jnp.float32),
            pltpu.SemaphoreType.DMA((NBUF, DCC)),
            pltpu.SemaphoreType.DMA((NPRE,)),
        ],
        compiler_params=pltpu.CompilerParams(
            dimension_semantics=("parallel", "arbitrary"),
        ),
    )(xcat, codebook, We1, be1, We2, be2, Wd1, bd1, Wd2, bd2, eps)


def kernel(x, codebook, pseudo_inputs, We1, be1, We2, be2, Wd1, bd1, Wd2, bd2, eps):
    xr = x.reshape(B, DIM_CODES, EMB).transpose(0, 2, 1).reshape(B, W_DIM)
    pr = pseudo_inputs.reshape(N_PSEUDO, W_DIM)
    xcat = jnp.concatenate(
        [xr, pr, jnp.zeros((N_PAD - N_ROWS, W_DIM), jnp.float32)], axis=0)
    enc2, z2, cwr2, cw_dist, idx2 = _run(
        xcat, codebook,
        We1, be1.reshape(1, -1), We2, be2.reshape(1, -1),
        Wd1, bd1.reshape(1, -1), Wd2, bd2.reshape(1, -1), eps)
    enc = enc2[0]
    mu = enc[:B, :Z_DIM]
    log_var = enc[:B, Z_DIM:]
    pseudo_mu = enc[B:N_ROWS, :Z_DIM]
    pseudo_log_var = enc[B:N_ROWS, Z_DIM:]
    idx = jnp.concatenate([idx2[i] for i in range(NC)], axis=1)
    return (mu, log_var, pseudo_mu, pseudo_log_var, z2[0], cwr2[0], cw_dist,
            idx.reshape(B, DIM_CODES, 1))
